# Initial kernel scaffold; baseline (speedup 1.0000x reference)
import jax
import jax.numpy as jnp
from jax.experimental import pallas as pl


def kernel(embeddings, centroids):
    raise NotImplementedError("write your pallas kernel here")


# speedup vs baseline: 2.4838x; 2.4838x over previous
import functools

import jax
import jax.numpy as jnp
from jax import lax
from jax.experimental import pallas as pl
from jax.experimental.pallas import tpu as pltpu

N = 32768
D = 128
K = 512
BN = 1024
NBLK = N // BN


def _body(a_ref, c_ref, idx_ref, loss_ref):
    a = a_ref[...]
    c = c_ref[...]
    prod = lax.dot_general(a, c, (((1,), (1,)), ((), ())),
                           preferred_element_type=jnp.float32)
    a2 = jnp.sum(a * a, axis=1, keepdims=True)
    c2 = jnp.sum(c * c, axis=1)[None, :]
    d2 = a2 + c2 - 2.0 * prod
    idx = jnp.argmin(d2, axis=1).astype(jnp.int32)
    idx_ref[0, 0, :] = idx
    m = jnp.maximum(jnp.min(d2, axis=1), 0.0)
    part = jnp.sum(m)

    @pl.when(pl.program_id(0) == 0)
    def _init():
        loss_ref[0, 0] = 0.0

    loss_ref[0, 0] += part


@jax.jit
def _run(embeddings, centroids):
    idx3, loss = pl.pallas_call(
        _body,
        grid=(NBLK,),
        in_specs=[
            pl.BlockSpec((BN, D), lambda i: (i, 0)),
            pl.BlockSpec((K, D), lambda i: (0, 0)),
        ],
        out_specs=[
            pl.BlockSpec((1, 1, BN), lambda i: (i, 0, 0)),
            pl.BlockSpec((1, 1), lambda i: (0, 0), memory_space=pltpu.SMEM),
        ],
        out_shape=[
            jax.ShapeDtypeStruct((NBLK, 1, BN), jnp.int32),
            jax.ShapeDtypeStruct((1, 1), jnp.float32),
        ],
    )(embeddings, centroids)
    return idx3, loss


def kernel(embeddings, centroids):
    idx3, loss = _run(embeddings, centroids)
    return (loss[0, 0] / (N * D), idx3.reshape(N))


# speedup vs baseline: 7.1972x; 2.8977x over previous
import jax
import jax.numpy as jnp
from jax import lax
from jax.experimental import pallas as pl
from jax.experimental.pallas import tpu as pltpu

N = 32768
D = 128
K = 512
BN = 4096
NBLK = N // BN


def _body(a_ref, c_ref, idx_ref, loss_ref, cm_ref, acc_ref):
    @pl.when(pl.program_id(0) == 0)
    def _prep():
        c0 = c_ref[...]
        cm_ref[...] = -0.5 * jnp.sum(c0 * c0, axis=1, keepdims=True)
        acc_ref[0] = 0.0

    a = a_ref[...]
    u = lax.dot_general(c_ref[...], a, (((1,), (1,)), ((), ())),
                        preferred_element_type=jnp.float32) + cm_ref[...]
    m = jnp.max(u, axis=0, keepdims=True)
    row = lax.broadcasted_iota(jnp.int32, (K, BN), 0)
    idx = jnp.min(jnp.where(u >= m, row, K), axis=0)
    idx_ref[...] = idx.astype(jnp.int32)
    acc_ref[0] += jnp.sum(a * a) - 2.0 * jnp.sum(m)

    @pl.when(pl.program_id(0) == NBLK - 1)
    def _fin():
        loss_ref[0] = acc_ref[0] * (1.0 / (N * D))


@jax.jit
def _run(embeddings, centroids):
    idx, loss = pl.pallas_call(
        _body,
        grid=(NBLK,),
        in_specs=[
            pl.BlockSpec((BN, D), lambda i: (i, 0)),
            pl.BlockSpec((K, D), lambda i: (0, 0)),
        ],
        out_specs=[
            pl.BlockSpec((BN,), lambda i: (i,)),
            pl.BlockSpec(memory_space=pltpu.SMEM),
        ],
        out_shape=[
            jax.ShapeDtypeStruct((N,), jnp.int32),
            jax.ShapeDtypeStruct((1,), jnp.float32),
        ],
        scratch_shapes=[
            pltpu.VMEM((K, 1), jnp.float32),
            pltpu.SMEM((1,), jnp.float32),
        ],
    )(embeddings, centroids)
    return idx, loss


def kernel(embeddings, centroids):
    idx, loss = _run(embeddings, centroids)
    return (loss.reshape(()), idx)


# speedup vs baseline: 8.1483x; 1.1321x over previous
import jax
import jax.numpy as jnp
from jax import lax
from jax.experimental import pallas as pl
from jax.experimental.pallas import tpu as pltpu

N = 32768
D = 128
K = 512
BN = 8192
NBLK = N // BN


def _body(a_ref, c_ref, idx_ref, loss_ref, cm_ref, acc_ref):
    @pl.when(pl.program_id(0) == 0)
    def _prep():
        c0 = c_ref[...]
        cm_ref[...] = -0.5 * jnp.sum(c0 * c0, axis=1, keepdims=True)
        acc_ref[0] = 0.0

    a = a_ref[...]
    u = lax.dot_general(c_ref[...], a, (((1,), (1,)), ((), ())),
                        preferred_element_type=jnp.float32) + cm_ref[...]
    m = jnp.max(u, axis=0, keepdims=True)
    row = lax.broadcasted_iota(jnp.int32, (K, BN), 0).astype(jnp.float32)
    idx = jnp.min(jnp.where(u >= m, row, float(K)), axis=0)
    idx_ref[...] = idx.astype(jnp.int32)
    acc_ref[0] += jnp.sum(a * a) - 2.0 * jnp.sum(m)

    @pl.when(pl.program_id(0) == NBLK - 1)
    def _fin():
        loss_ref[0] = acc_ref[0] * (1.0 / (N * D))


@jax.jit
def _run(embeddings, centroids):
    idx, loss = pl.pallas_call(
        _body,
        grid=(NBLK,),
        in_specs=[
            pl.BlockSpec((BN, D), lambda i: (i, 0)),
            pl.BlockSpec((K, D), lambda i: (0, 0)),
        ],
        out_specs=[
            pl.BlockSpec((BN,), lambda i: (i,)),
            pl.BlockSpec(memory_space=pltpu.SMEM),
        ],
        out_shape=[
            jax.ShapeDtypeStruct((N,), jnp.int32),
            jax.ShapeDtypeStruct((1,), jnp.float32),
        ],
        scratch_shapes=[
            pltpu.VMEM((K, 1), jnp.float32),
            pltpu.SMEM((1,), jnp.float32),
        ],
    )(embeddings, centroids)
    return idx, loss


def kernel(embeddings, centroids):
    idx, loss = _run(embeddings, centroids)
    return (loss.reshape(()), idx)
